# baseline (speedup 1.0000x reference)
import functools

import jax
import jax.numpy as jnp
from jax import lax
from jax.experimental import pallas as pl
from jax.experimental.pallas import tpu as pltpu
from jax.experimental.pallas import tpu_sc as plsc

N_NODES = 10000
N_EDGES = 320000
D = 128

NUM_CORES = 2
NUM_SUBCORES = 16
NUM_TILES = NUM_CORES * NUM_SUBCORES

K = 128
N_CHUNKS = 80
E_PAD = NUM_TILES * N_CHUNKS * K
TOT_CHUNKS = E_PAD // K
NBUF = 3
EBUF = 4
EPREF = 2

ROW_TILES = 10
ROWS_PER_TILE = N_NODES // ROW_TILES

_mesh = plsc.VectorSubcoreMesh(core_axis_name="c", subcore_axis_name="s")


def _lane_broadcast(vec, lane):
    idx = jnp.full((16, 1), lane, jnp.int32)
    return lax.gather(
        vec, idx,
        lax.GatherDimensionNumbers(
            offset_dims=(), collapsed_slice_dims=(0,), start_index_map=(0,)),
        slice_sizes=(1,),
        mode=lax.GatherScatterMode.PROMISE_IN_BOUNDS)


@functools.partial(
    pl.kernel,
    mesh=_mesh,
    out_type=jax.ShapeDtypeStruct((NUM_CORES, N_NODES, D), jnp.float32),
    scratch_types=[
        pltpu.VMEM((EBUF, 2, K), jnp.int32),
        pltpu.VMEM((EBUF * K,), jnp.float32),
        pltpu.VMEM((NBUF * K, D), jnp.float32),
        pltpu.VMEM_SHARED((N_NODES, D), jnp.float32),
        pltpu.SemaphoreType.DMA,
        pltpu.SemaphoreType.DMA,
        pltpu.SemaphoreType.DMA,
    ],
)
def _sc_aggregate(e2_hbm, w_hbm, feat_hbm, out_hbm,
                  e2_v, w_v, rows_v, h_sh, esem, gsem, ssem):
    cid = lax.axis_index("c")
    sid = lax.axis_index("s")
    tbase = (cid * NUM_SUBCORES + sid) * N_CHUNKS
    r0 = sid * ROWS_PER_TILE

    zeros16 = jnp.zeros((16,), jnp.float32)

    def _zero_body(i, _):
        rows_v[i // 8, pl.ds((i % 8) * 16, 16)] = zeros16
        return _

    lax.fori_loop(0, NBUF * K * (D // 16), _zero_body, None)

    @pl.when(sid < ROW_TILES)
    def _zero_h():
        for p in range(ROWS_PER_TILE // (NBUF * K)):
            pltpu.sync_copy(rows_v, h_sh.at[pl.ds(r0 + p * (NBUF * K),
                                                  NBUF * K)])
        rem = ROWS_PER_TILE % (NBUF * K)
        if rem:
            pltpu.sync_copy(
                rows_v.at[pl.ds(0, rem)],
                h_sh.at[pl.ds(r0 + ROWS_PER_TILE - rem, rem)])

    plsc.subcore_barrier()

    def _issue_edges(c):
        pltpu.async_copy(e2_hbm.at[tbase + c], e2_v.at[c % EBUF], esem)
        pltpu.async_copy(w_hbm.at[tbase + c],
                         w_v.at[pl.ds((c % EBUF) * K, K)], esem)

    def _drain_edges():
        pltpu.make_async_copy(e2_hbm.at[0], e2_v.at[0], esem).wait()
        pltpu.make_async_copy(w_hbm.at[0], w_v.at[pl.ds(0, K)], esem).wait()

    def _issue_gather(c):
        b = (c % NBUF) * K
        pltpu.async_copy(feat_hbm.at[e2_v.at[c % EBUF, 0]],
                         rows_v.at[pl.ds(b, K)], gsem)

    def _drain_gather():
        pltpu.make_async_copy(feat_hbm.at[pl.ds(0, K)],
                              rows_v.at[pl.ds(0, K)], gsem).wait()

    def _drain_scatter():
        pltpu.make_async_copy(rows_v.at[pl.ds(0, K)],
                              h_sh.at[pl.ds(0, K)], ssem).wait()

    for c in range(EPREF):
        _issue_edges(c)
    _drain_edges()
    _issue_gather(0)

    def _chunk_body(j, _):
        b = (j % NBUF) * K

        @pl.when(j >= 2)
        def _free():
            _drain_scatter()

        @pl.when(j + EPREF < N_CHUNKS)
        def _pf_edges():
            _issue_edges(j + EPREF)

        @pl.when(j + 1 < N_CHUNKS)
        def _pf_gather():
            _drain_edges()
            _issue_gather(j + 1)

        _drain_gather()

        for q in range(K // 16):
            wblk = w_v[pl.ds((j % EBUF) * K + q * 16, 16)]
            for lane in range(16):
                e = b + q * 16 + lane
                w16 = _lane_broadcast(wblk, lane)
                for g in range(D // 16):
                    sl = rows_v[e, pl.ds(g * 16, 16)]
                    rows_v[e, pl.ds(g * 16, 16)] = sl * w16

        pltpu.async_copy(rows_v.at[pl.ds(b, K)],
                         h_sh.at[e2_v.at[j % EBUF, 1]], ssem, add=True)
        return _

    lax.fori_loop(0, N_CHUNKS, _chunk_body, None)

    for _ in range(min(2, N_CHUNKS)):
        _drain_scatter()
    plsc.subcore_barrier()

    @pl.when(sid < ROW_TILES)
    def _writeout():
        pltpu.sync_copy(h_sh.at[pl.ds(r0, ROWS_PER_TILE)],
                        out_hbm.at[cid, pl.ds(r0, ROWS_PER_TILE)])


def _tc_matmul_body(h2_ref, w_ref, b_ref, out_ref):
    h = h2_ref[0] + h2_ref[1]
    out_ref[...] = (
        jnp.dot(h, w_ref[...], preferred_element_type=jnp.float32) + b_ref[...]
    )


def kernel(feat, edge_weights, edge_index, W, b):
    pad = E_PAD - N_EDGES
    srcp = jnp.pad(edge_index[0], (0, pad)).reshape(TOT_CHUNKS, K)
    dstp = jnp.pad(edge_index[1], (0, pad)).reshape(TOT_CHUNKS, K)
    e2 = jnp.stack([srcp, dstp], axis=1)
    w = jnp.pad(edge_weights.reshape(N_EDGES), (0, pad)).reshape(TOT_CHUNKS, K)
    h2 = _sc_aggregate(e2, w, feat)
    rst = pl.pallas_call(
        _tc_matmul_body,
        out_shape=jax.ShapeDtypeStruct((N_NODES, D), jnp.float32),
    )(h2, W, b.reshape(1, D))
    return rst


# speedup vs baseline: 3.3712x; 3.3712x over previous
import functools

import jax
import jax.numpy as jnp
from jax import lax
from jax.experimental import pallas as pl
from jax.experimental.pallas import tpu as pltpu
from jax.experimental.pallas import tpu_sc as plsc

N_NODES = 10000
N_EDGES = 320000
D = 128

NUM_CORES = 2
NUM_SUBCORES = 16

E_PER_CORE = N_EDGES // NUM_CORES
E_PER_TILE = E_PER_CORE // NUM_SUBCORES
K = 80
N_CHUNKS = E_PER_TILE // K
NBUF = 4
DBUF = 6
PREF = 2
EPREF = 3

ROW_TILES = 10
ROWS_PER_TILE = N_NODES // ROW_TILES

_mesh = plsc.VectorSubcoreMesh(core_axis_name="c", subcore_axis_name="s")


def _lane_broadcast(vec, lane):
    idx = jnp.full((16, 1), lane, jnp.int32)
    return lax.gather(
        vec, idx,
        lax.GatherDimensionNumbers(
            offset_dims=(), collapsed_slice_dims=(0,), start_index_map=(0,)),
        slice_sizes=(1,),
        mode=lax.GatherScatterMode.PROMISE_IN_BOUNDS)


@functools.partial(
    pl.kernel,
    mesh=_mesh,
    out_type=jax.ShapeDtypeStruct((NUM_CORES, N_NODES, D), jnp.float32),
    scratch_types=[
        pltpu.VMEM((NBUF, K), jnp.int32),
        pltpu.VMEM((DBUF, K), jnp.int32),
        pltpu.VMEM((NBUF, K), jnp.float32),
        pltpu.VMEM((NBUF * K, D), jnp.float32),
        pltpu.VMEM_SHARED((N_NODES, D), jnp.float32),
        pltpu.SemaphoreType.DMA,
        pltpu.SemaphoreType.DMA,
        pltpu.SemaphoreType.DMA,
    ],
)
def _sc_aggregate(src_hbm, dst_hbm, w_hbm, feat_hbm, out_hbm,
                  src_v, dst_v, w_v, rows_v, h_sh, esem, gsem, ssem):
    cid = lax.axis_index("c")
    sid = lax.axis_index("s")
    ebase = cid * E_PER_CORE + sid * E_PER_TILE
    r0 = sid * ROWS_PER_TILE

    zeros16 = jnp.zeros((16,), jnp.float32)

    def _zero_body(i, _):
        rows_v[i // 8, pl.ds((i % 8) * 16, 16)] = zeros16
        return _

    lax.fori_loop(0, NBUF * K * (D // 16), _zero_body, None)

    @pl.when(sid < ROW_TILES)
    def _zero_h():
        for p in range(ROWS_PER_TILE // (NBUF * K)):
            pltpu.sync_copy(rows_v, h_sh.at[pl.ds(r0 + p * (NBUF * K),
                                                  NBUF * K)])
        rem = ROWS_PER_TILE % (NBUF * K)
        if rem:
            pltpu.sync_copy(
                rows_v.at[pl.ds(0, rem)],
                h_sh.at[pl.ds(r0 + ROWS_PER_TILE - rem, rem)])

    plsc.subcore_barrier()

    def _issue_edges(c):
        off = ebase + c * K
        pltpu.async_copy(src_hbm.at[pl.ds(off, K)], src_v.at[c % NBUF], esem)
        pltpu.async_copy(w_hbm.at[pl.ds(off, K)], w_v.at[c % NBUF], esem)
        pltpu.async_copy(dst_hbm.at[pl.ds(off, K)], dst_v.at[c % DBUF], esem)

    def _drain_edges():
        pltpu.make_async_copy(src_hbm.at[pl.ds(0, K)], src_v.at[0], esem).wait()
        pltpu.make_async_copy(w_hbm.at[pl.ds(0, K)], w_v.at[0], esem).wait()
        pltpu.make_async_copy(dst_hbm.at[pl.ds(0, K)], dst_v.at[0], esem).wait()

    def _issue_gather(c):
        b = (c % NBUF) * K
        pltpu.async_copy(feat_hbm.at[src_v.at[c % NBUF]],
                         rows_v.at[pl.ds(b, K)], gsem)

    def _drain_gather():
        pltpu.make_async_copy(feat_hbm.at[pl.ds(0, K)],
                              rows_v.at[pl.ds(0, K)], gsem).wait()

    def _drain_scatter():
        pltpu.make_async_copy(rows_v.at[pl.ds(0, K)],
                              h_sh.at[pl.ds(0, K)], ssem).wait()

    for c in range(EPREF):
        _issue_edges(c)
    for c in range(PREF):
        _drain_edges()
        _issue_gather(c)

    def _chunk_body(j, _):
        b = (j % NBUF) * K

        @pl.when(j >= NBUF - PREF)
        def _free():
            _drain_scatter()

        @pl.when(j + EPREF < N_CHUNKS)
        def _pf_edges():
            _issue_edges(j + EPREF)

        @pl.when(j + PREF < N_CHUNKS)
        def _pf_gather():
            _drain_edges()
            _issue_gather(j + PREF)

        _drain_gather()

        def _scale_body(q, _):
            wblk = w_v[j % NBUF, pl.ds(q * 16, 16)]
            for lane in range(16):
                e = b + q * 16 + lane
                w16 = _lane_broadcast(wblk, lane)
                for g in range(D // 16):
                    sl = rows_v[e, pl.ds(g * 16, 16)]
                    rows_v[e, pl.ds(g * 16, 16)] = sl * w16
            return _

        lax.fori_loop(0, K // 16, _scale_body, None)

        pltpu.async_copy(rows_v.at[pl.ds(b, K)], h_sh.at[dst_v.at[j % DBUF]],
                         ssem, add=True)
        return _

    lax.fori_loop(0, N_CHUNKS, _chunk_body, None)

    for _ in range(min(NBUF - PREF, N_CHUNKS)):
        _drain_scatter()
    plsc.subcore_barrier()

    @pl.when(sid < ROW_TILES)
    def _writeout():
        pltpu.sync_copy(h_sh.at[pl.ds(r0, ROWS_PER_TILE)],
                        out_hbm.at[cid, pl.ds(r0, ROWS_PER_TILE)])


def _tc_matmul_body(h2_ref, w_ref, b_ref, out_ref):
    h = h2_ref[0] + h2_ref[1]
    out_ref[...] = (
        jnp.dot(h, w_ref[...], preferred_element_type=jnp.float32) + b_ref[...]
    )


def kernel(feat, edge_weights, edge_index, W, b):
    src = edge_index[0]
    dst = edge_index[1]
    w = edge_weights.reshape(N_EDGES)
    h2 = _sc_aggregate(src, dst, w, feat)
    rst = pl.pallas_call(
        _tc_matmul_body,
        out_shape=jax.ShapeDtypeStruct((N_NODES, D), jnp.float32),
    )(h2, W, b.reshape(1, D))
    return rst


# speedup vs baseline: 3.4486x; 1.0230x over previous
import functools

import jax
import jax.numpy as jnp
from jax import lax
from jax.experimental import pallas as pl
from jax.experimental.pallas import tpu as pltpu
from jax.experimental.pallas import tpu_sc as plsc

N_NODES = 10000
N_EDGES = 320000
D = 128

NUM_CORES = 2
NUM_SUBCORES = 16

E_PER_CORE = N_EDGES // NUM_CORES
E_PER_TILE = E_PER_CORE // NUM_SUBCORES
K = 80
N_CHUNKS = E_PER_TILE // K
NBUF = 4
SBUF = 5
DBUF = 7
PREF = 2
EPREF = 4

ROW_TILES = 10
ROWS_PER_TILE = N_NODES // ROW_TILES

_mesh = plsc.VectorSubcoreMesh(core_axis_name="c", subcore_axis_name="s")


def _lane_broadcast(vec, lane):
    idx = jnp.full((16, 1), lane, jnp.int32)
    return lax.gather(
        vec, idx,
        lax.GatherDimensionNumbers(
            offset_dims=(), collapsed_slice_dims=(0,), start_index_map=(0,)),
        slice_sizes=(1,),
        mode=lax.GatherScatterMode.PROMISE_IN_BOUNDS)


@functools.partial(
    pl.kernel,
    mesh=_mesh,
    out_type=jax.ShapeDtypeStruct((NUM_CORES, N_NODES, D), jnp.float32),
    scratch_types=[
        pltpu.VMEM((SBUF, K), jnp.int32),
        pltpu.VMEM((DBUF, K), jnp.int32),
        pltpu.VMEM((SBUF, K), jnp.float32),
        pltpu.VMEM((NBUF * K, D), jnp.float32),
        pltpu.VMEM_SHARED((N_NODES, D), jnp.float32),
        pltpu.SemaphoreType.DMA,
        pltpu.SemaphoreType.DMA,
        pltpu.SemaphoreType.DMA,
    ],
)
def _sc_aggregate(src_hbm, dst_hbm, w_hbm, feat_hbm, out_hbm,
                  src_v, dst_v, w_v, rows_v, h_sh, esem, gsem, ssem):
    cid = lax.axis_index("c")
    sid = lax.axis_index("s")
    ebase = cid * E_PER_CORE + sid * E_PER_TILE
    r0 = sid * ROWS_PER_TILE

    zeros16 = jnp.zeros((16,), jnp.float32)

    def _zero_body(i, _):
        rows_v[i // 8, pl.ds((i % 8) * 16, 16)] = zeros16
        return _

    lax.fori_loop(0, NBUF * K * (D // 16), _zero_body, None)

    @pl.when(sid < ROW_TILES)
    def _zero_h():
        for p in range(ROWS_PER_TILE // (NBUF * K)):
            pltpu.sync_copy(rows_v, h_sh.at[pl.ds(r0 + p * (NBUF * K),
                                                  NBUF * K)])
        rem = ROWS_PER_TILE % (NBUF * K)
        if rem:
            pltpu.sync_copy(
                rows_v.at[pl.ds(0, rem)],
                h_sh.at[pl.ds(r0 + ROWS_PER_TILE - rem, rem)])

    plsc.subcore_barrier()

    def _issue_edges(c):
        off = ebase + c * K
        pltpu.async_copy(src_hbm.at[pl.ds(off, K)], src_v.at[c % SBUF], esem)
        pltpu.async_copy(w_hbm.at[pl.ds(off, K)], w_v.at[c % SBUF], esem)
        pltpu.async_copy(dst_hbm.at[pl.ds(off, K)], dst_v.at[c % DBUF], esem)

    def _drain_edges():
        pltpu.make_async_copy(src_hbm.at[pl.ds(0, K)], src_v.at[0], esem).wait()
        pltpu.make_async_copy(w_hbm.at[pl.ds(0, K)], w_v.at[0], esem).wait()
        pltpu.make_async_copy(dst_hbm.at[pl.ds(0, K)], dst_v.at[0], esem).wait()

    def _issue_gather(c):
        b = (c % NBUF) * K
        pltpu.async_copy(feat_hbm.at[src_v.at[c % SBUF]],
                         rows_v.at[pl.ds(b, K)], gsem)

    def _drain_gather():
        pltpu.make_async_copy(feat_hbm.at[pl.ds(0, K)],
                              rows_v.at[pl.ds(0, K)], gsem).wait()

    def _drain_scatter():
        pltpu.make_async_copy(rows_v.at[pl.ds(0, K)],
                              h_sh.at[pl.ds(0, K)], ssem).wait()

    for c in range(EPREF):
        _issue_edges(c)
    for c in range(PREF):
        _drain_edges()
        _issue_gather(c)

    def _chunk_body(j, _):
        b = (j % NBUF) * K

        @pl.when(j >= NBUF - PREF)
        def _free():
            _drain_scatter()

        @pl.when(j + EPREF < N_CHUNKS)
        def _pf_edges():
            _issue_edges(j + EPREF)

        @pl.when(j + PREF < N_CHUNKS)
        def _pf_gather():
            _drain_edges()
            _issue_gather(j + PREF)

        _drain_gather()

        def _scale_body(q, _):
            wblk = w_v[j % SBUF, pl.ds(q * 16, 16)]
            for lane in range(16):
                e = b + q * 16 + lane
                w16 = _lane_broadcast(wblk, lane)
                for g in range(D // 16):
                    sl = rows_v[e, pl.ds(g * 16, 16)]
                    rows_v[e, pl.ds(g * 16, 16)] = sl * w16
            return _

        lax.fori_loop(0, K // 16, _scale_body, None)

        pltpu.async_copy(rows_v.at[pl.ds(b, K)], h_sh.at[dst_v.at[j % DBUF]],
                         ssem, add=True)
        return _

    lax.fori_loop(0, N_CHUNKS, _chunk_body, None)

    for _ in range(min(NBUF - PREF, N_CHUNKS)):
        _drain_scatter()
    plsc.subcore_barrier()

    @pl.when(sid < ROW_TILES)
    def _writeout():
        pltpu.sync_copy(h_sh.at[pl.ds(r0, ROWS_PER_TILE)],
                        out_hbm.at[cid, pl.ds(r0, ROWS_PER_TILE)])


def _tc_matmul_body(h2_ref, w_ref, b_ref, out_ref):
    h = h2_ref[0] + h2_ref[1]
    out_ref[...] = (
        jnp.dot(h, w_ref[...], preferred_element_type=jnp.float32) + b_ref[...]
    )


def kernel(feat, edge_weights, edge_index, W, b):
    src = edge_index[0]
    dst = edge_index[1]
    w = edge_weights.reshape(N_EDGES)
    h2 = _sc_aggregate(src, dst, w, feat)
    rst = pl.pallas_call(
        _tc_matmul_body,
        out_shape=jax.ShapeDtypeStruct((N_NODES, D), jnp.float32),
    )(h2, W, b.reshape(1, D))
    return rst
